# baseline (speedup 1.0000x reference)
import functools

import jax
import jax.numpy as jnp
from jax.experimental import pallas as pl
from jax.experimental.pallas import tpu as pltpu


def _p2_body(points2_ref, Wf0_ref, b_ref, g_ref, bet_ref, out_ref):
    z = jnp.dot(points2_ref[...], Wf0_ref[...],
                preferred_element_type=jnp.float32) + b_ref[...]
    m = jnp.mean(z, axis=0, keepdims=True)
    v = jnp.mean(z * z, axis=0, keepdims=True) - m * m
    out_ref[...] = (z - m) * (g_ref[...] * jax.lax.rsqrt(v + 1e-5)) + bet_ref[...]


def _top3(e, n2):
    bq = e.shape[0]
    lanes = 128
    nch = n2 // lanes
    lane = jax.lax.broadcasted_iota(jnp.int32, (bq, lanes), 1)
    inf = jnp.float32(jnp.inf)
    bigi = jnp.int32(n2)
    v1 = e[:, :lanes]
    c1 = lane
    v2 = jnp.full((bq, lanes), inf, jnp.float32)
    c2 = jnp.full((bq, lanes), bigi, jnp.int32)
    v3 = v2
    c3 = c2
    for k in range(1, nch):
        nv = e[:, k * lanes:(k + 1) * lanes]
        ncol = lane + k * lanes
        lt1 = nv < v1
        lt2 = nv < v2
        lt3 = nv < v3
        v3 = jnp.where(lt2, v2, jnp.where(lt3, nv, v3))
        c3 = jnp.where(lt2, c2, jnp.where(lt3, ncol, c3))
        v2 = jnp.where(lt1, v1, jnp.where(lt2, nv, v2))
        c2 = jnp.where(lt1, c1, jnp.where(lt2, ncol, c2))
        v1 = jnp.where(lt1, nv, v1)
        c1 = jnp.where(lt1, ncol, c1)
    vals, idxs = [], []
    for _ in range(3):
        m = jnp.min(jnp.minimum(jnp.minimum(v1, v2), v3), axis=1, keepdims=True)
        i = jnp.minimum(
            jnp.minimum(
                jnp.min(jnp.where(v1 == m, c1, bigi), axis=1, keepdims=True),
                jnp.min(jnp.where(v2 == m, c2, bigi), axis=1, keepdims=True)),
            jnp.min(jnp.where(v3 == m, c3, bigi), axis=1, keepdims=True))
        vals.append(m)
        idxs.append(i)
        s1 = c1 == i
        s2 = c2 == i
        s3 = c3 == i
        v1 = jnp.where(s1, inf, v1)
        c1 = jnp.where(s1, bigi, c1)
        v2 = jnp.where(s2, inf, v2)
        c2 = jnp.where(s2, bigi, c2)
        v3 = jnp.where(s3, inf, v3)
        c3 = jnp.where(s3, bigi, c3)
    return vals, idxs


def _weights3_direct(vals):
    rs = [1.0 / (jnp.sqrt(jnp.maximum(m, 0.0)) + 1e-8) for m in vals]
    rsum = rs[0] + rs[1] + rs[2]
    return [r / rsum for r in rs]


def _knn_interp_body(caug_ref, x2aug_ref, p2_ref, interp_ref, *, n2):
    ca = caug_ref[...]
    c = ca[:, 0:3]
    m2x2t = x2aug_ref[...][0:3, :]
    cc = jnp.sum(c * c, axis=1, keepdims=True)
    sq2 = x2aug_ref[...][3:4, :]
    e = cc + sq2 + jnp.dot(c, m2x2t, preferred_element_type=jnp.float32)
    vals, idxs = _top3(e, n2)
    ws = _weights3_direct(vals)
    iota = jax.lax.broadcasted_iota(jnp.int32, e.shape, 1)
    wmat = (jnp.where(iota == idxs[0], ws[0], 0.0)
            + jnp.where(iota == idxs[1], ws[1], 0.0)
            + jnp.where(iota == idxs[2], ws[2], 0.0))
    interp_ref[...] = jnp.dot(wmat, p2_ref[...], preferred_element_type=jnp.float32)


def _xstat_body(x_ref, sum_ref, gram_ref):
    j = pl.program_id(0)
    x = x_ref[...]

    @pl.when(j == 0)
    def _():
        sum_ref[...] = jnp.zeros_like(sum_ref)
        gram_ref[...] = jnp.zeros_like(gram_ref)

    sum_ref[...] += jnp.sum(x, axis=0, keepdims=True)
    gram_ref[...] += jax.lax.dot_general(
        x, x, (((0,), (0,)), ((), ())), preferred_element_type=jnp.float32)


def _mlp_body(x_ref, interp_ref, Ws0_ref, W1_ref, ss_ref, sh_ref, b1_ref,
              y_ref, ysum_ref, ysq_ref):
    j = pl.program_id(0)
    s = jnp.dot(x_ref[...], Ws0_ref[...], preferred_element_type=jnp.float32)
    skip = s * ss_ref[...] + sh_ref[...]
    h = jnp.maximum(interp_ref[...] + skip, 0.0)
    y = jnp.dot(h, W1_ref[...], preferred_element_type=jnp.float32) + b1_ref[...]
    y_ref[...] = y

    @pl.when(j == 0)
    def _():
        ysum_ref[...] = jnp.zeros_like(ysum_ref)
        ysq_ref[...] = jnp.zeros_like(ysq_ref)

    ysum_ref[...] += jnp.sum(y, axis=0, keepdims=True)
    ysq_ref[...] += jnp.sum(y * y, axis=0, keepdims=True)


def _finish_body(y_ref, ys_ref, yh_ref, out_ref):
    out_ref[...] = jnp.maximum(y_ref[...] * ys_ref[...] + yh_ref[...], 0.0)


def kernel(xyz1, points1, offset1, xyz2, points2, offset2,
           Wf0, bf0, gf0, betf0, Ws0, bs0, gs0, bets0, W1, b1, g1, bet1):
    n1, _ = xyz1.shape
    n2, prev = points2.shape
    skipd = points1.shape[1]
    m0 = Wf0.shape[1]
    m1 = W1.shape[1]
    f32 = jnp.float32

    row = lambda v: v.reshape(1, -1).astype(f32)

    p2 = pl.pallas_call(
        _p2_body,
        out_shape=jax.ShapeDtypeStruct((n2, m0), f32),
    )(points2, Wf0, row(bf0), row(gf0), row(betf0))

    bq = 256
    nb = n1 // bq
    caug = jnp.concatenate([xyz1, jnp.ones((n1, 1), f32)], axis=1)
    sq2 = jnp.sum(xyz2 * xyz2, axis=1)[None, :]
    x2aug = jnp.concatenate([-2.0 * xyz2.T, sq2], axis=0)
    interp = pl.pallas_call(
        functools.partial(_knn_interp_body, n2=n2),
        grid=(nb,),
        in_specs=[
            pl.BlockSpec((bq, 4), lambda j: (j, 0)),
            pl.BlockSpec((4, n2), lambda j: (0, 0)),
            pl.BlockSpec((n2, m0), lambda j: (0, 0)),
        ],
        out_specs=pl.BlockSpec((bq, m0), lambda j: (j, 0)),
        out_shape=jax.ShapeDtypeStruct((n1, m0), f32),
    )(caug, x2aug, p2)

    bs = 2048
    nbs = n1 // bs
    xsum, xgram = pl.pallas_call(
        _xstat_body,
        grid=(nbs,),
        in_specs=[pl.BlockSpec((bs, skipd), lambda j: (j, 0))],
        out_specs=[
            pl.BlockSpec((1, skipd), lambda j: (0, 0)),
            pl.BlockSpec((skipd, skipd), lambda j: (0, 0)),
        ],
        out_shape=[
            jax.ShapeDtypeStruct((1, skipd), f32),
            jax.ShapeDtypeStruct((skipd, skipd), f32),
        ],
    )(points1)

    mean_x = xsum / n1
    mean_s = mean_x @ Ws0 + bs0[None, :]
    aw = (xgram / n1) @ Ws0
    e_s2 = (jnp.sum(Ws0 * aw, axis=0, keepdims=True)
            + 2.0 * bs0[None, :] * (mean_x @ Ws0) + bs0[None, :] ** 2)
    var_s = e_s2 - mean_s * mean_s
    sscale = gs0[None, :] * jax.lax.rsqrt(var_s + 1e-5)
    sshift = (bs0[None, :] - mean_s) * sscale + bets0[None, :]

    bm = 2048
    nbm = n1 // bm
    y, ysum, ysq = pl.pallas_call(
        _mlp_body,
        grid=(nbm,),
        in_specs=[
            pl.BlockSpec((bm, skipd), lambda j: (j, 0)),
            pl.BlockSpec((bm, m0), lambda j: (j, 0)),
            pl.BlockSpec((skipd, m0), lambda j: (0, 0)),
            pl.BlockSpec((m0, m1), lambda j: (0, 0)),
            pl.BlockSpec((1, m0), lambda j: (0, 0)),
            pl.BlockSpec((1, m0), lambda j: (0, 0)),
            pl.BlockSpec((1, m1), lambda j: (0, 0)),
        ],
        out_specs=[
            pl.BlockSpec((bm, m1), lambda j: (j, 0)),
            pl.BlockSpec((1, m1), lambda j: (0, 0)),
            pl.BlockSpec((1, m1), lambda j: (0, 0)),
        ],
        out_shape=[
            jax.ShapeDtypeStruct((n1, m1), f32),
            jax.ShapeDtypeStruct((1, m1), f32),
            jax.ShapeDtypeStruct((1, m1), f32),
        ],
    )(points1, interp, Ws0, W1, sscale.astype(f32), sshift.astype(f32), row(b1))

    mean_y = ysum / n1
    var_y = ysq / n1 - mean_y * mean_y
    yscale = g1[None, :] * jax.lax.rsqrt(var_y + 1e-5)
    yshift = bet1[None, :] - mean_y * yscale

    bf = 2048
    nbf = n1 // bf
    out = pl.pallas_call(
        _finish_body,
        grid=(nbf,),
        in_specs=[
            pl.BlockSpec((bf, m1), lambda j: (j, 0)),
            pl.BlockSpec((1, m1), lambda j: (0, 0)),
            pl.BlockSpec((1, m1), lambda j: (0, 0)),
        ],
        out_specs=pl.BlockSpec((bf, m1), lambda j: (j, 0)),
        out_shape=jax.ShapeDtypeStruct((n1, m1), f32),
    )(y, yscale.astype(f32), yshift.astype(f32))
    return out


# speedup vs baseline: 1.0398x; 1.0398x over previous
import functools

import jax
import jax.numpy as jnp
from jax.experimental import pallas as pl
from jax.experimental.pallas import tpu as pltpu
from jax.experimental.pallas import tpu_sc as plsc

_NC = 2
_NS = 16
_NW = _NC * _NS


def _p2_body(points2_ref, Wf0_ref, b_ref, g_ref, bet_ref, out_ref):
    z = jnp.dot(points2_ref[...], Wf0_ref[...],
                preferred_element_type=jnp.float32) + b_ref[...]
    m = jnp.mean(z, axis=0, keepdims=True)
    v = jnp.mean(z * z, axis=0, keepdims=True) - m * m
    out_ref[...] = (z - m) * (g_ref[...] * jax.lax.rsqrt(v + 1e-5)) + bet_ref[...]


def _top3(dot, sq2, n2):
    bq = dot.shape[0]
    lanes = 128
    nch = n2 // lanes
    lane = jax.lax.broadcasted_iota(jnp.int32, (bq, lanes), 1)
    inf = jnp.float32(jnp.inf)
    bigi = jnp.int32(n2)
    v1 = dot[:, :lanes] + sq2[:, :lanes]
    c1 = lane
    v2 = jnp.full((bq, lanes), inf, jnp.float32)
    c2 = jnp.full((bq, lanes), bigi, jnp.int32)
    v3 = v2
    c3 = c2
    for k in range(1, nch):
        nv = dot[:, k * lanes:(k + 1) * lanes] + sq2[:, k * lanes:(k + 1) * lanes]
        ncol = lane + k * lanes
        lt1 = nv < v1
        lt2 = nv < v2
        lt3 = nv < v3
        v3 = jnp.where(lt2, v2, jnp.where(lt3, nv, v3))
        c3 = jnp.where(lt2, c2, jnp.where(lt3, ncol, c3))
        v2 = jnp.where(lt1, v1, jnp.where(lt2, nv, v2))
        c2 = jnp.where(lt1, c1, jnp.where(lt2, ncol, c2))
        v1 = jnp.where(lt1, nv, v1)
        c1 = jnp.where(lt1, ncol, c1)
    m1 = jnp.min(v1, axis=1, keepdims=True)
    i1 = jnp.min(jnp.where(v1 == m1, c1, bigi), axis=1, keepdims=True)
    s1 = c1 == i1
    v1 = jnp.where(s1, inf, v1)
    c1 = jnp.where(s1, bigi, c1)
    m2 = jnp.min(jnp.minimum(v1, v2), axis=1, keepdims=True)
    i2 = jnp.minimum(
        jnp.min(jnp.where(v1 == m2, c1, bigi), axis=1, keepdims=True),
        jnp.min(jnp.where(v2 == m2, c2, bigi), axis=1, keepdims=True))
    s1 = c1 == i2
    s2 = c2 == i2
    v1 = jnp.where(s1, inf, v1)
    c1 = jnp.where(s1, bigi, c1)
    v2 = jnp.where(s2, inf, v2)
    c2 = jnp.where(s2, bigi, c2)
    m3 = jnp.min(jnp.minimum(jnp.minimum(v1, v2), v3), axis=1, keepdims=True)
    i3 = jnp.minimum(
        jnp.minimum(
            jnp.min(jnp.where(v1 == m3, c1, bigi), axis=1, keepdims=True),
            jnp.min(jnp.where(v2 == m3, c2, bigi), axis=1, keepdims=True)),
        jnp.min(jnp.where(v3 == m3, c3, bigi), axis=1, keepdims=True))
    return [m1, m2, m3], [i1, i2, i3]


def _knn_body(caug_ref, x2aug_ref, idx_ref, w_ref, *, n2):
    ca = caug_ref[...]
    c = ca[:, 0:3]
    m2x2t = x2aug_ref[...][0:3, :]
    sq2 = x2aug_ref[...][3:4, :]
    dot = jnp.dot(c, m2x2t, preferred_element_type=jnp.float32)
    vals, idxs = _top3(dot, sq2, n2)
    cc = jnp.sum(c * c, axis=1, keepdims=True)
    rs = [1.0 / (jnp.sqrt(jnp.maximum(m + cc, 0.0)) + 1e-8) for m in vals]
    rsum = rs[0] + rs[1] + rs[2]
    idx_ref[...] = jnp.concatenate(idxs, axis=1)
    w_ref[...] = jnp.concatenate([r / rsum for r in rs], axis=1)


def _sc_gather_call(p2, idxf, chunk=128):
    b = idxf.shape[0]
    d = p2.shape[1]
    bpw = b // _NW
    nch = bpw // chunk
    mesh = plsc.VectorSubcoreMesh(core_axis_name="c", subcore_axis_name="s")

    def body(p2_hbm, idx_hbm, out_hbm, idx_v, rows_v, sem):
        wid = jax.lax.axis_index("s") * _NC + jax.lax.axis_index("c")
        base = wid * bpw
        for t in range(nch):
            off = base + t * chunk
            pltpu.sync_copy(idx_hbm.at[pl.ds(off, chunk)], idx_v)
            pltpu.async_copy(p2_hbm.at[idx_v], rows_v, sem).wait()
            pltpu.sync_copy(rows_v, out_hbm.at[pl.ds(off, chunk)])

    f = pl.kernel(
        body,
        out_type=jax.ShapeDtypeStruct((b, d), jnp.float32),
        mesh=mesh,
        scratch_types=[
            pltpu.VMEM((chunk,), jnp.int32),
            pltpu.VMEM((chunk, d), jnp.float32),
            pltpu.SemaphoreType.DMA,
        ],
    )
    return f(p2, idxf)


def _xstat_body(x_ref, sum_ref, gram_ref):
    j = pl.program_id(0)
    x = x_ref[...]

    @pl.when(j == 0)
    def _():
        sum_ref[...] = jnp.zeros_like(sum_ref)
        gram_ref[...] = jnp.zeros_like(gram_ref)

    sum_ref[...] += jnp.sum(x, axis=0, keepdims=True)
    gram_ref[...] += jax.lax.dot_general(
        x, x, (((0,), (0,)), ((), ())), preferred_element_type=jnp.float32)


def _mlp_body(x_ref, g_ref, w_ref, Ws0_ref, W1_ref, ss_ref, sh_ref, b1_ref,
              y_ref, ysum_ref, ysq_ref):
    j = pl.program_id(0)
    g = g_ref[...]
    w = w_ref[...]
    interp = g[0] * w[:, 0:1] + g[1] * w[:, 1:2] + g[2] * w[:, 2:3]
    s = jnp.dot(x_ref[...], Ws0_ref[...], preferred_element_type=jnp.float32)
    skip = s * ss_ref[...] + sh_ref[...]
    h = jnp.maximum(interp + skip, 0.0)
    y = jnp.dot(h, W1_ref[...], preferred_element_type=jnp.float32) + b1_ref[...]
    y_ref[...] = y

    @pl.when(j == 0)
    def _():
        ysum_ref[...] = jnp.zeros_like(ysum_ref)
        ysq_ref[...] = jnp.zeros_like(ysq_ref)

    ysum_ref[...] += jnp.sum(y, axis=0, keepdims=True)
    ysq_ref[...] += jnp.sum(y * y, axis=0, keepdims=True)


def _finish_body(y_ref, ys_ref, yh_ref, out_ref):
    out_ref[...] = jnp.maximum(y_ref[...] * ys_ref[...] + yh_ref[...], 0.0)


def kernel(xyz1, points1, offset1, xyz2, points2, offset2,
           Wf0, bf0, gf0, betf0, Ws0, bs0, gs0, bets0, W1, b1, g1, bet1):
    n1, _ = xyz1.shape
    n2, prev = points2.shape
    skipd = points1.shape[1]
    m0 = Wf0.shape[1]
    m1 = W1.shape[1]
    f32 = jnp.float32

    row = lambda v: v.reshape(1, -1).astype(f32)

    p2 = pl.pallas_call(
        _p2_body,
        out_shape=jax.ShapeDtypeStruct((n2, m0), f32),
    )(points2, Wf0, row(bf0), row(gf0), row(betf0))

    bq = 256
    nb = n1 // bq
    caug = jnp.concatenate([xyz1, jnp.ones((n1, 1), f32)], axis=1)
    sq2 = jnp.sum(xyz2 * xyz2, axis=1)[None, :]
    x2aug = jnp.concatenate([-2.0 * xyz2.T, sq2], axis=0)
    idx, w = pl.pallas_call(
        functools.partial(_knn_body, n2=n2),
        grid=(nb,),
        in_specs=[
            pl.BlockSpec((bq, 4), lambda j: (j, 0)),
            pl.BlockSpec((4, n2), lambda j: (0, 0)),
        ],
        out_specs=[
            pl.BlockSpec((bq, 3), lambda j: (j, 0)),
            pl.BlockSpec((bq, 3), lambda j: (j, 0)),
        ],
        out_shape=[
            jax.ShapeDtypeStruct((n1, 3), jnp.int32),
            jax.ShapeDtypeStruct((n1, 3), f32),
        ],
    )(caug, x2aug)

    idxf = idx.T.reshape(-1)
    g = _sc_gather_call(p2, idxf).reshape(3, n1, m0)

    bs = 2048
    nbs = n1 // bs
    xsum, xgram = pl.pallas_call(
        _xstat_body,
        grid=(nbs,),
        in_specs=[pl.BlockSpec((bs, skipd), lambda j: (j, 0))],
        out_specs=[
            pl.BlockSpec((1, skipd), lambda j: (0, 0)),
            pl.BlockSpec((skipd, skipd), lambda j: (0, 0)),
        ],
        out_shape=[
            jax.ShapeDtypeStruct((1, skipd), f32),
            jax.ShapeDtypeStruct((skipd, skipd), f32),
        ],
    )(points1)

    mean_x = xsum / n1
    mean_s = mean_x @ Ws0 + bs0[None, :]
    aw = (xgram / n1) @ Ws0
    e_s2 = (jnp.sum(Ws0 * aw, axis=0, keepdims=True)
            + 2.0 * bs0[None, :] * (mean_x @ Ws0) + bs0[None, :] ** 2)
    var_s = e_s2 - mean_s * mean_s
    sscale = gs0[None, :] * jax.lax.rsqrt(var_s + 1e-5)
    sshift = (bs0[None, :] - mean_s) * sscale + bets0[None, :]

    bm = 2048
    nbm = n1 // bm
    y, ysum, ysq = pl.pallas_call(
        _mlp_body,
        grid=(nbm,),
        in_specs=[
            pl.BlockSpec((bm, skipd), lambda j: (j, 0)),
            pl.BlockSpec((3, bm, m0), lambda j: (0, j, 0)),
            pl.BlockSpec((bm, 3), lambda j: (j, 0)),
            pl.BlockSpec((skipd, m0), lambda j: (0, 0)),
            pl.BlockSpec((m0, m1), lambda j: (0, 0)),
            pl.BlockSpec((1, m0), lambda j: (0, 0)),
            pl.BlockSpec((1, m0), lambda j: (0, 0)),
            pl.BlockSpec((1, m1), lambda j: (0, 0)),
        ],
        out_specs=[
            pl.BlockSpec((bm, m1), lambda j: (j, 0)),
            pl.BlockSpec((1, m1), lambda j: (0, 0)),
            pl.BlockSpec((1, m1), lambda j: (0, 0)),
        ],
        out_shape=[
            jax.ShapeDtypeStruct((n1, m1), f32),
            jax.ShapeDtypeStruct((1, m1), f32),
            jax.ShapeDtypeStruct((1, m1), f32),
        ],
    )(points1, g, w, Ws0, W1, sscale.astype(f32), sshift.astype(f32), row(b1))

    mean_y = ysum / n1
    var_y = ysq / n1 - mean_y * mean_y
    yscale = g1[None, :] * jax.lax.rsqrt(var_y + 1e-5)
    yshift = bet1[None, :] - mean_y * yscale

    bf = 2048
    nbf = n1 // bf
    out = pl.pallas_call(
        _finish_body,
        grid=(nbf,),
        in_specs=[
            pl.BlockSpec((bf, m1), lambda j: (j, 0)),
            pl.BlockSpec((1, m1), lambda j: (0, 0)),
            pl.BlockSpec((1, m1), lambda j: (0, 0)),
        ],
        out_specs=pl.BlockSpec((bf, m1), lambda j: (j, 0)),
        out_shape=jax.ShapeDtypeStruct((n1, m1), f32),
    )(y, yscale.astype(f32), yshift.astype(f32))
    return out
